# Initial kernel scaffold; baseline (speedup 1.0000x reference)
import jax
import jax.numpy as jnp
from jax import lax
from jax.experimental import pallas as pl
from jax.experimental.pallas import tpu as pltpu
from jax.experimental.pallas import tpu_sc as plsc

N_DST = 10000
D = 128
B = 4
E = 320000
NS = 16
L = 16
TE = E // NS
K = 80
G = TE // K
NP = 10240
STRIPE = NP // NS
ROWS_PER_TILE = N_DST // NS
WB = 125
ZQ = ROWS_PER_TILE // WB


def _body(x_hbm, src_hbm, dst_hbm, w_hbm, out_hbm,
          src_v, dst_v, w_v, table_v, tmp_v, tmp2_v, buf0, buf1, zbuf,
          tables_sh, norm_sh, acc_sh, sem0, sem1):
    cid = lax.axis_index("c")
    sid = lax.axis_index("s")
    row0 = sid * G
    eb = sid * TE

    pltpu.sync_copy(src_hbm.at[pl.ds(row0, G)], src_v)
    pltpu.sync_copy(dst_hbm.at[pl.ds(row0, G)], dst_v)
    pltpu.sync_copy(w_hbm.at[pl.ds(eb, TE)], w_v)

    @pl.loop(0, NP // L)
    def _zero_table(i):
        table_v[pl.ds(i * L, L)] = jnp.zeros((L,), jnp.float32)

    @pl.loop(0, G)
    def _p1(g):
        for k5 in range(K // L):
            off = g * K + k5 * L
            wv = w_v[pl.ds(off, L)]
            wg = jnp.exp(-0.5 * wv * wv)
            w_v[pl.ds(off, L)] = wg
            dv = dst_v[g, pl.ds(k5 * L, L)]
            plsc.addupdate_scatter(table_v, [dv], wg)

    pltpu.sync_copy(table_v, tables_sh.at[sid])
    plsc.subcore_barrier()
    st = sid * STRIPE
    pltpu.sync_copy(tables_sh.at[0, pl.ds(st, STRIPE)], tmp_v)
    for j in range(1, NS):
        pltpu.sync_copy(tables_sh.at[j, pl.ds(st, STRIPE)], tmp2_v)

        @pl.loop(0, STRIPE // L)
        def _acc(i):
            tmp_v[pl.ds(i * L, L)] = (tmp_v[pl.ds(i * L, L)]
                                      + tmp2_v[pl.ds(i * L, L)])
    pltpu.sync_copy(tmp_v, norm_sh.at[pl.ds(st, STRIPE)])
    plsc.subcore_barrier()
    pltpu.sync_copy(norm_sh, table_v)

    @pl.loop(0, G)
    def _p1c(g):
        for k5 in range(K // L):
            off = g * K + k5 * L
            dv = dst_v[g, pl.ds(k5 * L, L)]
            nv = plsc.load_gather(table_v, [dv])
            w_v[pl.ds(off, L)] = w_v[pl.ds(off, L)] / (nv + 1e-8)

    @pl.loop(0, WB)
    def _zz(r):
        for c8 in range(D // L):
            zbuf[r, pl.ds(c8 * L, L)] = jnp.zeros((L,), jnp.float32)

    bufs = (buf0, buf1)
    sems = (sem0, sem1)

    for b in range(B):
        @pl.when(cid == (b // 2))
        def _batch():
            xb = x_hbm.at[b]
            for q in range(ZQ):
                pltpu.sync_copy(
                    zbuf, acc_sh.at[pl.ds(sid * ROWS_PER_TILE + q * WB, WB)])
            plsc.subcore_barrier()

            for slot in range(2):
                pltpu.async_copy(xb.at[src_v.at[slot]], bufs[slot], sems[slot])

            @pl.loop(0, G, step=2)
            def _main(g):
                for slot in range(2):
                    gc = g + slot
                    pltpu.make_async_copy(
                        xb.at[pl.ds(0, K)], bufs[slot], sems[slot]).wait()

                    @pl.loop(0, K, unroll=8)
                    def _scale(r):
                        s = w_v[gc * K + r]
                        for c8 in range(D // L):
                            bufs[slot][r, pl.ds(c8 * L, L)] = (
                                bufs[slot][r, pl.ds(c8 * L, L)] * s)

                    pltpu.sync_copy(bufs[slot], acc_sh.at[dst_v.at[gc]],
                                    add=True)
                    nxt = gc + 2

                    @pl.when(nxt < G)
                    def _prefetch():
                        pltpu.async_copy(xb.at[src_v.at[nxt]], bufs[slot],
                                         sems[slot])

            plsc.subcore_barrier()
            for q in range(ZQ):
                r0 = sid * ROWS_PER_TILE + q * WB
                pltpu.sync_copy(acc_sh.at[pl.ds(r0, WB)],
                                out_hbm.at[b].at[pl.ds(r0, WB)])
            plsc.subcore_barrier()


def kernel(x, edge_index, weights):
    src2d = edge_index[0].reshape(NS * G, K)
    dst2d = edge_index[1].reshape(NS * G, K)
    mesh = plsc.VectorSubcoreMesh(core_axis_name="c", subcore_axis_name="s")
    f = pl.kernel(
        _body,
        out_type=jax.ShapeDtypeStruct((B, N_DST, D), jnp.float32),
        mesh=mesh,
        scratch_types=[
            pltpu.VMEM((G, K), jnp.int32),
            pltpu.VMEM((G, K), jnp.int32),
            pltpu.VMEM((TE,), jnp.float32),
            pltpu.VMEM((NP,), jnp.float32),
            pltpu.VMEM((STRIPE,), jnp.float32),
            pltpu.VMEM((STRIPE,), jnp.float32),
            pltpu.VMEM((K, D), jnp.float32),
            pltpu.VMEM((K, D), jnp.float32),
            pltpu.VMEM((WB, D), jnp.float32),
            pltpu.VMEM_SHARED((NS, NP), jnp.float32),
            pltpu.VMEM_SHARED((NP,), jnp.float32),
            pltpu.VMEM_SHARED((N_DST, D), jnp.float32),
            pltpu.SemaphoreType.DMA,
            pltpu.SemaphoreType.DMA,
        ],
    )
    return f(x, src2d, dst2d, weights)


# speedup vs baseline: 38.7069x; 38.7069x over previous
import jax
import jax.numpy as jnp
from jax import lax
from jax.experimental import pallas as pl
from jax.experimental.pallas import tpu as pltpu
from jax.experimental.pallas import tpu_sc as plsc

N_DST = 10000
D = 128
DP = 64
NPASS = D // DP
B = 4
E = 320000
NS = 16
L = 16
TE = E // NS
K = 80
G = TE // K
NP = 10240
STRIPE = NP // NS
ZCH = 80
NCH = N_DST // ZCH
KCH = -(-NCH // NS)


def _body(x_hbm, src_hbm, dst_hbm, w_hbm, out_hbm,
          src_v, dst_v, w_v, table_v, tmp_v, idx_v, buf0, buf1, zbuf,
          norm_sh, acc_sh, sem0, sem1):
    cid = lax.axis_index("c")
    sid = lax.axis_index("s")
    eb = sid * TE
    NR = NP // L
    SR = NR // NS

    pltpu.sync_copy(src_hbm.at[sid], src_v)
    pltpu.sync_copy(dst_hbm.at[sid], dst_v)
    pltpu.sync_copy(w_hbm.at[pl.ds(eb, TE)], w_v)

    @pl.loop(0, NR)
    def _zero_table(i):
        table_v[i] = jnp.zeros((L,), jnp.float32)

    for k in range(NR // K):
        for m in range(K // L):
            idx_v[k, pl.ds(m * L, L)] = (lax.iota(jnp.int32, 16)
                                         + (k * K + m * L))

    @pl.loop(0, SR)
    def _zero_tmp(i):
        tmp_v[i] = jnp.zeros((L,), jnp.float32)

    @pl.loop(0, G)
    def _p1(g):
        for k5 in range(K // L):
            off = g * K + k5 * L
            wv = w_v[pl.ds(off, L)]
            wg = jnp.exp(-0.5 * wv * wv)
            w_v[pl.ds(off, L)] = wg
            dv = dst_v[g, pl.ds(k5 * L, L)]
            plsc.addupdate_scatter(
                table_v, [jnp.right_shift(dv, 4), jnp.bitwise_and(dv, 15)],
                wg)

    pltpu.sync_copy(tmp_v, norm_sh.at[pl.ds(sid * SR, SR)])
    plsc.subcore_barrier()
    for k in range(NR // K):
        pltpu.sync_copy(table_v.at[pl.ds(k * K, K)],
                        norm_sh.at[idx_v.at[k]], add=True)
    plsc.subcore_barrier()
    pltpu.sync_copy(norm_sh, table_v)

    @pl.loop(0, G)
    def _p1c(g):
        for k5 in range(K // L):
            off = g * K + k5 * L
            dv = dst_v[g, pl.ds(k5 * L, L)]
            nv = plsc.load_gather(
                table_v, [jnp.right_shift(dv, 4), jnp.bitwise_and(dv, 15)])
            w_v[pl.ds(off, L)] = w_v[pl.ds(off, L)] / (nv + 1e-8)

    @pl.loop(0, ZCH)
    def _zz(r):
        for c8 in range(DP // L):
            zbuf[r, pl.ds(c8 * L, L)] = jnp.zeros((L,), jnp.float32)

    bufs = (buf0, buf1)
    sems = (sem0, sem1)

    for b in range(B):
        for p in range(NPASS):
            @pl.when(cid == (b // 2))
            def _batch():
                xb = x_hbm.at[b].at[p]

                @pl.loop(0, KCH)
                def _z(k):
                    c = sid + NS * k

                    @pl.when(c < NCH)
                    def _zc():
                        pltpu.sync_copy(zbuf, acc_sh.at[pl.ds(c * ZCH, ZCH)])
                plsc.subcore_barrier()

                for slot in range(2):
                    pltpu.async_copy(xb.at[src_v.at[slot]], bufs[slot],
                                     sems[slot])

                @pl.loop(0, G, step=2)
                def _main(g):
                    for slot in range(2):
                        gc = g + slot
                        pltpu.make_async_copy(
                            xb.at[pl.ds(0, K)], bufs[slot], sems[slot]).wait()

                        @pl.loop(0, K // L)
                        def _scale(r16):
                            w16 = w_v[pl.ds(gc * K + r16 * L, L)]
                            for j in range(L):
                                s = w16[j]
                                r = r16 * L + j
                                for c8 in range(DP // L):
                                    bufs[slot][r, pl.ds(c8 * L, L)] = (
                                        bufs[slot][r, pl.ds(c8 * L, L)] * s)

                        pltpu.sync_copy(bufs[slot], acc_sh.at[dst_v.at[gc]],
                                        add=True)
                        nxt = gc + 2

                        @pl.when(nxt < G)
                        def _prefetch():
                            pltpu.async_copy(xb.at[src_v.at[nxt]], bufs[slot],
                                             sems[slot])

                plsc.subcore_barrier()

                @pl.loop(0, KCH)
                def _wb(k):
                    c = sid + NS * k

                    @pl.when(c < NCH)
                    def _wbc():
                        pltpu.sync_copy(
                            acc_sh.at[pl.ds(c * ZCH, ZCH)],
                            out_hbm.at[b].at[p].at[pl.ds(c * ZCH, ZCH)])
                plsc.subcore_barrier()


def kernel(x, edge_index, weights):
    src2d = edge_index[0].reshape(NS, G, K)
    dst2d = edge_index[1].reshape(NS, G, K)
    xt = jnp.swapaxes(x.reshape(B, -1, NPASS, DP), 1, 2)
    mesh = plsc.VectorSubcoreMesh(core_axis_name="c", subcore_axis_name="s")
    f = pl.kernel(
        _body,
        out_type=jax.ShapeDtypeStruct((B, NPASS, N_DST, DP), jnp.float32),
        mesh=mesh,
        compiler_params=pltpu.CompilerParams(needs_layout_passes=False,
                                             use_tc_tiling_on_sc=False),
        scratch_types=[
            pltpu.VMEM((G, K), jnp.int32),
            pltpu.VMEM((G, K), jnp.int32),
            pltpu.VMEM((TE,), jnp.float32),
            pltpu.VMEM((NP // L, L), jnp.float32),
            pltpu.VMEM((NP // L // NS, L), jnp.float32),
            pltpu.VMEM((NP // L // K, K), jnp.int32),
            pltpu.VMEM((K, DP), jnp.float32),
            pltpu.VMEM((K, DP), jnp.float32),
            pltpu.VMEM((ZCH, DP), jnp.float32),
            pltpu.VMEM_SHARED((NP // L, L), jnp.float32),
            pltpu.VMEM_SHARED((N_DST, DP), jnp.float32),
            pltpu.SemaphoreType.DMA,
            pltpu.SemaphoreType.DMA,
        ],
    )
    out4 = f(xt, src2d, dst2d, weights)
    return jnp.swapaxes(out4, 1, 2).reshape(B, N_DST, D)


# speedup vs baseline: 42.6128x; 1.1009x over previous
import jax
import jax.numpy as jnp
from jax import lax
from jax.experimental import pallas as pl
from jax.experimental.pallas import tpu as pltpu
from jax.experimental.pallas import tpu_sc as plsc

N_DST = 10000
D = 128
DP = 64
NPASS = D // DP
B = 4
E = 320000
NS = 16
L = 16
TE = E // NS
K = 80
G = TE // K
NP = 10240
STRIPE = NP // NS
ZCH = 80
NCH = N_DST // ZCH
KCH = -(-NCH // NS)


def _body(x_hbm, src_hbm, dst_hbm, w_hbm, out_hbm,
          src_v, dst_v, w_v, table_v, tmp_v, idx_v, buf0, buf1, buf2,
          norm_sh, acc_sh, gsem0, gsem1, gsem2, ssem0, ssem1, ssem2):
    cid = lax.axis_index("c")
    sid = lax.axis_index("s")
    eb = sid * TE
    NR = NP // L
    SR = NR // NS

    pltpu.sync_copy(src_hbm.at[sid], src_v)
    pltpu.sync_copy(dst_hbm.at[sid], dst_v)
    pltpu.sync_copy(w_hbm.at[pl.ds(eb, TE)], w_v)

    @pl.loop(0, NR)
    def _zero_table(i):
        table_v[i] = jnp.zeros((L,), jnp.float32)

    for k in range(NR // K):
        for m in range(K // L):
            idx_v[k, pl.ds(m * L, L)] = (lax.iota(jnp.int32, 16)
                                         + (k * K + m * L))

    @pl.loop(0, SR)
    def _zero_tmp(i):
        tmp_v[i] = jnp.zeros((L,), jnp.float32)

    @pl.loop(0, G)
    def _p1(g):
        for k5 in range(K // L):
            off = g * K + k5 * L
            wv = w_v[pl.ds(off, L)]
            wg = jnp.exp(-0.5 * wv * wv)
            w_v[pl.ds(off, L)] = wg
            dv = dst_v[g, pl.ds(k5 * L, L)]
            plsc.addupdate_scatter(
                table_v, [jnp.right_shift(dv, 4), jnp.bitwise_and(dv, 15)],
                wg)

    pltpu.sync_copy(tmp_v, norm_sh.at[pl.ds(sid * SR, SR)])
    plsc.subcore_barrier()
    for k in range(NR // K):
        pltpu.sync_copy(table_v.at[pl.ds(k * K, K)],
                        norm_sh.at[idx_v.at[k]], add=True)
    plsc.subcore_barrier()
    pltpu.sync_copy(norm_sh, table_v)

    @pl.loop(0, G)
    def _p1c(g):
        for k5 in range(K // L):
            off = g * K + k5 * L
            dv = dst_v[g, pl.ds(k5 * L, L)]
            nv = plsc.load_gather(
                table_v, [jnp.right_shift(dv, 4), jnp.bitwise_and(dv, 15)])
            w_v[pl.ds(off, L)] = w_v[pl.ds(off, L)] / (nv + 1e-8)

    bufs = (buf0, buf1, buf2)
    gsems = (gsem0, gsem1, gsem2)
    ssems = (ssem0, ssem1, ssem2)
    NB = 3

    for b in range(B):
        for p in range(NPASS):
            @pl.when(cid == (b // 2))
            def _batch():
                xb = x_hbm.at[b].at[p]

                @pl.loop(0, ZCH)
                def _zz(r):
                    for c8 in range(DP // L):
                        buf0[r, pl.ds(c8 * L, L)] = jnp.zeros((L,),
                                                              jnp.float32)

                @pl.loop(0, KCH)
                def _z(k):
                    c = sid + NS * k

                    @pl.when(c < NCH)
                    def _zc():
                        pltpu.sync_copy(buf0, acc_sh.at[pl.ds(c * ZCH, ZCH)])
                plsc.subcore_barrier()

                for slot in range(NB):
                    pltpu.async_copy(xb.at[src_v.at[slot]], bufs[slot],
                                     gsems[slot])

                def _chunk(gc, slot, do_prefetch):
                    pltpu.make_async_copy(
                        xb.at[pl.ds(0, K)], bufs[slot], gsems[slot]).wait()

                    @pl.loop(0, K // L)
                    def _scale(r16):
                        w16 = w_v[pl.ds(gc * K + r16 * L, L)]
                        for j in range(L):
                            s = w16[j]
                            r = r16 * L + j
                            for c8 in range(DP // L):
                                bufs[slot][r, pl.ds(c8 * L, L)] = (
                                    bufs[slot][r, pl.ds(c8 * L, L)] * s)

                    pltpu.async_copy(bufs[slot], acc_sh.at[dst_v.at[gc]],
                                     ssems[slot], add=True)
                    if do_prefetch:
                        nxt = gc + NB

                        @pl.when(nxt < G)
                        def _prefetch():
                            pltpu.make_async_copy(
                                bufs[slot], acc_sh.at[pl.ds(0, K)],
                                ssems[slot]).wait()
                            pltpu.async_copy(xb.at[src_v.at[nxt]], bufs[slot],
                                             gsems[slot])

                GM = (G // NB) * NB

                @pl.loop(0, GM, step=NB)
                def _main(g):
                    for slot in range(NB):
                        _chunk(g + slot, slot, True)

                for gc in range(GM, G):
                    _chunk(gc, gc % NB, False)

                for slot in range(NB):
                    pltpu.make_async_copy(bufs[slot], acc_sh.at[pl.ds(0, K)],
                                          ssems[slot]).wait()

                plsc.subcore_barrier()

                @pl.loop(0, KCH)
                def _wb(k):
                    c = sid + NS * k

                    @pl.when(c < NCH)
                    def _wbc():
                        pltpu.sync_copy(
                            acc_sh.at[pl.ds(c * ZCH, ZCH)],
                            out_hbm.at[b].at[p].at[pl.ds(c * ZCH, ZCH)])
                plsc.subcore_barrier()


def kernel(x, edge_index, weights):
    src2d = edge_index[0].reshape(NS, G, K)
    dst2d = edge_index[1].reshape(NS, G, K)
    xt = jnp.swapaxes(x.reshape(B, -1, NPASS, DP), 1, 2)
    mesh = plsc.VectorSubcoreMesh(core_axis_name="c", subcore_axis_name="s")
    f = pl.kernel(
        _body,
        out_type=jax.ShapeDtypeStruct((B, NPASS, N_DST, DP), jnp.float32),
        mesh=mesh,
        compiler_params=pltpu.CompilerParams(needs_layout_passes=False,
                                             use_tc_tiling_on_sc=False),
        scratch_types=[
            pltpu.VMEM((G, K), jnp.int32),
            pltpu.VMEM((G, K), jnp.int32),
            pltpu.VMEM((TE,), jnp.float32),
            pltpu.VMEM((NP // L, L), jnp.float32),
            pltpu.VMEM((NP // L // NS, L), jnp.float32),
            pltpu.VMEM((NP // L // K, K), jnp.int32),
            pltpu.VMEM((K, DP), jnp.float32),
            pltpu.VMEM((K, DP), jnp.float32),
            pltpu.VMEM((K, DP), jnp.float32),
            pltpu.VMEM_SHARED((NP // L, L), jnp.float32),
            pltpu.VMEM_SHARED((N_DST, DP), jnp.float32),
            pltpu.SemaphoreType.DMA,
            pltpu.SemaphoreType.DMA,
            pltpu.SemaphoreType.DMA,
            pltpu.SemaphoreType.DMA,
            pltpu.SemaphoreType.DMA,
            pltpu.SemaphoreType.DMA,
        ],
    )
    out4 = f(xt, src2d, dst2d, weights)
    return jnp.swapaxes(out4, 1, 2).reshape(B, N_DST, D)


# speedup vs baseline: 63.6567x; 1.4938x over previous
import jax
import jax.numpy as jnp
from jax import lax
from jax.experimental import pallas as pl
from jax.experimental.pallas import tpu as pltpu
from jax.experimental.pallas import tpu_sc as plsc

N_DST = 10000
D = 128
B = 4
E = 320000
NS = 16
L = 16
TE = E // NS
K = 80
G = TE // K
GRP = 10
GRPK = GRP * K
NGRP = G // GRP
NEB = 3
NB = 2
NP = 10240
NR = NP // L
SR = NR // NS
ZCH = 80
NCH = N_DST // ZCH
KCH = -(-NCH // NS)


def _body(x_hbm, src_hbm, dst_hbm, w_hbm, out_hbm,
          esrc_v, edst_v, ew_v, table_v, tmp_v, idxr_v, idx2_v, buf0, buf1,
          norm_sh, acc_sh, esem, gsem0, gsem1, ssem0, ssem1):
    cid = lax.axis_index("c")
    sid = lax.axis_index("s")
    eb = sid * TE

    def issue_egroup(grp, with_src):
        off = eb + grp * GRPK
        dst_off = lax.rem(grp, NEB) * GRPK
        if with_src:
            pltpu.async_copy(src_hbm.at[pl.ds(off, GRPK)],
                             esrc_v.at[pl.ds(dst_off, GRPK)], esem)
        pltpu.async_copy(dst_hbm.at[pl.ds(off, GRPK)],
                         edst_v.at[pl.ds(dst_off, GRPK)], esem)
        pltpu.async_copy(w_hbm.at[pl.ds(off, GRPK)],
                         ew_v.at[pl.ds(dst_off, GRPK)], esem)

    def wait_egroup(with_src):
        if with_src:
            pltpu.make_async_copy(src_hbm.at[pl.ds(0, GRPK)],
                                  esrc_v.at[pl.ds(0, GRPK)], esem).wait()
        pltpu.make_async_copy(dst_hbm.at[pl.ds(0, GRPK)],
                              edst_v.at[pl.ds(0, GRPK)], esem).wait()
        pltpu.make_async_copy(w_hbm.at[pl.ds(0, GRPK)],
                              ew_v.at[pl.ds(0, GRPK)], esem).wait()

    @pl.loop(0, NR)
    def _zero_table(i):
        table_v[i] = jnp.zeros((L,), jnp.float32)

    for k in range(NR // K):
        for m in range(K // L):
            idxr_v[k, pl.ds(m * L, L)] = (lax.iota(jnp.int32, 16)
                                          + (k * K + m * L))

    @pl.loop(0, SR)
    def _zero_tmp(i):
        tmp_v[i] = jnp.zeros((L,), jnp.float32)

    for g in range(NEB):
        issue_egroup(g, False)

    @pl.loop(0, NGRP)
    def _p1(grp):
        ebuf = lax.rem(grp, NEB) * GRPK
        wait_egroup(False)

        @pl.loop(0, GRPK // L)
        def _s(m):
            off = ebuf + m * L
            wv = ew_v[pl.ds(off, L)]
            wg = jnp.exp(-0.5 * wv * wv)
            dv = edst_v[pl.ds(off, L)]
            plsc.addupdate_scatter(
                table_v, [jnp.right_shift(dv, 4), jnp.bitwise_and(dv, 15)],
                wg)

        nxt = grp + NEB

        @pl.when(nxt < NGRP)
        def _issue():
            issue_egroup(nxt, False)

    pltpu.sync_copy(tmp_v, norm_sh.at[pl.ds(sid * SR, SR)])
    plsc.subcore_barrier()
    for k in range(NR // K):
        pltpu.sync_copy(table_v.at[pl.ds(k * K, K)],
                        norm_sh.at[idxr_v.at[k]], add=True)
    plsc.subcore_barrier()
    pltpu.sync_copy(norm_sh, table_v)

    bufs = (buf0, buf1)
    gsems = (gsem0, gsem1)
    ssems = (ssem0, ssem1)

    RING = NEB * GRP

    @pl.loop(0, B // 2)
    def _batch(bi):
        b = cid * (B // 2) + bi
        xb = x_hbm.at[b]

        @pl.loop(0, ZCH)
        def _zz(r):
            for c8 in range(D // L):
                buf0[r, pl.ds(c8 * L, L)] = jnp.zeros((L,), jnp.float32)

        @pl.loop(0, KCH)
        def _z(k):
            c = sid + NS * k

            @pl.when(c < NCH)
            def _zc():
                pltpu.sync_copy(buf0, acc_sh.at[pl.ds(c * ZCH, ZCH)])
        plsc.subcore_barrier()

        issue_egroup(0, True)
        issue_egroup(1, True)
        wait_egroup(True)
        for slot in range(NB):
            pltpu.async_copy(xb.at[esrc_v.at[pl.ds(slot * K, K)]],
                             bufs[slot], gsems[slot])

        @pl.loop(0, NGRP)
        def _g2(grp):
            @pl.when(grp + 2 < NGRP)
            def _issue():
                issue_egroup(grp + 2, True)

            @pl.when(grp + 1 < NGRP)
            def _wait():
                wait_egroup(True)

            @pl.loop(0, GRP, step=NB)
            def _pair(cc0):
                for slot in range(NB):
                    gc = grp * GRP + cc0 + slot
                    ring = lax.rem(gc, RING)
                    coff = ring * K
                    noff = lax.rem(gc + NB, RING) * K

                    pltpu.make_async_copy(xb.at[pl.ds(0, K)], bufs[slot],
                                          gsems[slot]).wait()

                    @pl.loop(0, K // L)
                    def _norm(m):
                        dv = edst_v[pl.ds(coff + m * L, L)]
                        wv = ew_v[pl.ds(coff + m * L, L)]
                        wg = jnp.exp(-0.5 * wv * wv)
                        nv = plsc.load_gather(
                            table_v,
                            [jnp.right_shift(dv, 4), jnp.bitwise_and(dv, 15)])
                        ew_v[pl.ds(coff + m * L, L)] = wg / (nv + 1e-8)
                        idx2_v[ring, pl.ds(m * L, L)] = dv

                    @pl.loop(0, K // L)
                    def _scale(r16):
                        w16 = ew_v[pl.ds(coff + r16 * L, L)]
                        for j in range(L):
                            s = w16[j]
                            r = r16 * L + j
                            for c8 in range(D // L):
                                bufs[slot][r, pl.ds(c8 * L, L)] = (
                                    bufs[slot][r, pl.ds(c8 * L, L)] * s)

                    pltpu.async_copy(bufs[slot], acc_sh.at[idx2_v.at[ring]],
                                     ssems[slot], add=True)

                    @pl.when(gc + NB < G)
                    def _prefetch():
                        pltpu.make_async_copy(bufs[slot],
                                              acc_sh.at[pl.ds(0, K)],
                                              ssems[slot]).wait()
                        pltpu.async_copy(xb.at[esrc_v.at[pl.ds(noff, K)]],
                                         bufs[slot], gsems[slot])

        for slot in range(NB):
            pltpu.make_async_copy(bufs[slot], acc_sh.at[pl.ds(0, K)],
                                  ssems[slot]).wait()

        plsc.subcore_barrier()

        @pl.loop(0, KCH)
        def _wb(k):
            c = sid + NS * k

            @pl.when(c < NCH)
            def _wbc():
                pltpu.sync_copy(acc_sh.at[pl.ds(c * ZCH, ZCH)],
                                out_hbm.at[b].at[pl.ds(c * ZCH, ZCH)])
        plsc.subcore_barrier()


def kernel(x, edge_index, weights):
    src1d = edge_index[0]
    dst1d = edge_index[1]
    mesh = plsc.VectorSubcoreMesh(core_axis_name="c", subcore_axis_name="s")
    f = pl.kernel(
        _body,
        out_type=jax.ShapeDtypeStruct((B, N_DST, D), jnp.float32),
        mesh=mesh,
        compiler_params=pltpu.CompilerParams(needs_layout_passes=False,
                                             use_tc_tiling_on_sc=False),
        scratch_types=[
            pltpu.VMEM((NEB * GRPK,), jnp.int32),
            pltpu.VMEM((NEB * GRPK,), jnp.int32),
            pltpu.VMEM((NEB * GRPK,), jnp.float32),
            pltpu.VMEM((NP // L, L), jnp.float32),
            pltpu.VMEM((SR, L), jnp.float32),
            pltpu.VMEM((NR // K, K), jnp.int32),
            pltpu.VMEM((NEB * GRP, K), jnp.int32),
            pltpu.VMEM((K, D), jnp.float32),
            pltpu.VMEM((K, D), jnp.float32),
            pltpu.VMEM_SHARED((NP // L, L), jnp.float32),
            pltpu.VMEM_SHARED((N_DST, D), jnp.float32),
            pltpu.SemaphoreType.DMA,
            pltpu.SemaphoreType.DMA,
            pltpu.SemaphoreType.DMA,
            pltpu.SemaphoreType.DMA,
            pltpu.SemaphoreType.DMA,
        ],
    )
    return f(x, src1d, dst1d, weights)


# speedup vs baseline: 63.9845x; 1.0052x over previous
import jax
import jax.numpy as jnp
from jax import lax
from jax.experimental import pallas as pl
from jax.experimental.pallas import tpu as pltpu
from jax.experimental.pallas import tpu_sc as plsc

N_DST = 10000
D = 128
B = 4
E = 320000
NS = 16
L = 16
TE = E // NS
K = 80
G = TE // K
GRP = 10
GRPK = GRP * K
NGRP = G // GRP
NEB = 3
NB = 2
NP = 10240
NR = NP // L
SR = NR // NS
ZCH = 80
NCH = N_DST // ZCH
KCH = -(-NCH // NS)


def _body(x_hbm, src_hbm, dst_hbm, w_hbm, out_hbm,
          esrc_v, edst_v, ew_v, table_v, tmp_v, idxr_v, idx2_v, buf0, buf1,
          norm_sh, acc_sh, esem, gsem0, gsem1, ssem0, ssem1):
    cid = lax.axis_index("c")
    sid = lax.axis_index("s")
    eb = sid * TE

    def issue_egroup(grp, with_src):
        off = eb + grp * GRPK
        dst_off = lax.rem(grp, NEB) * GRPK
        if with_src:
            pltpu.async_copy(src_hbm.at[pl.ds(off, GRPK)],
                             esrc_v.at[pl.ds(dst_off, GRPK)], esem)
        pltpu.async_copy(dst_hbm.at[pl.ds(off, GRPK)],
                         edst_v.at[pl.ds(dst_off, GRPK)], esem)
        pltpu.async_copy(w_hbm.at[pl.ds(off, GRPK)],
                         ew_v.at[pl.ds(dst_off, GRPK)], esem)

    def wait_egroup(with_src):
        if with_src:
            pltpu.make_async_copy(src_hbm.at[pl.ds(0, GRPK)],
                                  esrc_v.at[pl.ds(0, GRPK)], esem).wait()
        pltpu.make_async_copy(dst_hbm.at[pl.ds(0, GRPK)],
                              edst_v.at[pl.ds(0, GRPK)], esem).wait()
        pltpu.make_async_copy(w_hbm.at[pl.ds(0, GRPK)],
                              ew_v.at[pl.ds(0, GRPK)], esem).wait()

    @pl.loop(0, NR)
    def _zero_table(i):
        table_v[i] = jnp.zeros((L,), jnp.float32)

    for k in range(NR // K):
        for m in range(K // L):
            idxr_v[k, pl.ds(m * L, L)] = (lax.iota(jnp.int32, 16)
                                          + (k * K + m * L))

    @pl.loop(0, SR)
    def _zero_tmp(i):
        tmp_v[i] = jnp.zeros((L,), jnp.float32)

    for g in range(NEB):
        issue_egroup(g, False)

    @pl.loop(0, NGRP)
    def _p1(grp):
        ebuf = lax.rem(grp, NEB) * GRPK
        wait_egroup(False)

        @pl.loop(0, GRPK // L)
        def _s(m):
            off = ebuf + m * L
            wv = ew_v[pl.ds(off, L)]
            wg = jnp.exp(-0.5 * wv * wv)
            dv = edst_v[pl.ds(off, L)]
            plsc.addupdate_scatter(
                table_v, [jnp.right_shift(dv, 4), jnp.bitwise_and(dv, 15)],
                wg)

        nxt = grp + NEB

        @pl.when(nxt < NGRP)
        def _issue():
            issue_egroup(nxt, False)

    pltpu.sync_copy(tmp_v, norm_sh.at[pl.ds(sid * SR, SR)])
    plsc.subcore_barrier()
    for k in range(NR // K):
        pltpu.sync_copy(table_v.at[pl.ds(k * K, K)],
                        norm_sh.at[idxr_v.at[k]], add=True)
    plsc.subcore_barrier()
    pltpu.sync_copy(norm_sh, table_v)

    bufs = (buf0, buf1)
    gsems = (gsem0, gsem1)
    ssems = (ssem0, ssem1)

    RING = NEB * GRP

    @pl.loop(0, B // 2)
    def _batch(bi):
        b = cid * (B // 2) + bi
        xb = x_hbm.at[b]

        @pl.loop(0, ZCH)
        def _zz(r):
            for c8 in range(D // L):
                buf0[r, pl.ds(c8 * L, L)] = jnp.zeros((L,), jnp.float32)

        @pl.loop(0, KCH)
        def _z(k):
            c = sid + NS * k

            @pl.when(c < NCH)
            def _zc():
                pltpu.sync_copy(buf0, acc_sh.at[pl.ds(c * ZCH, ZCH)])
        plsc.subcore_barrier()

        issue_egroup(0, True)
        issue_egroup(1, True)
        wait_egroup(True)
        for slot in range(NB):
            pltpu.async_copy(xb.at[esrc_v.at[pl.ds(slot * K, K)]],
                             bufs[slot], gsems[slot])

        @pl.loop(0, NGRP)
        def _g2(grp):
            @pl.when(grp + 2 < NGRP)
            def _issue():
                issue_egroup(grp + 2, True)

            @pl.when(grp + 1 < NGRP)
            def _wait():
                wait_egroup(True)

            @pl.loop(0, GRP, step=NB)
            def _pair(cc0):
                for slot in range(NB):
                    gc = grp * GRP + cc0 + slot
                    ring = lax.rem(gc, RING)
                    coff = ring * K
                    noff = lax.rem(gc + NB, RING) * K

                    pltpu.make_async_copy(xb.at[pl.ds(0, K)], bufs[slot],
                                          gsems[slot]).wait()

                    @pl.loop(0, K // L)
                    def _norm(m):
                        dv = edst_v[pl.ds(coff + m * L, L)]
                        wv = ew_v[pl.ds(coff + m * L, L)]
                        ew_v[pl.ds(coff + m * L, L)] = jnp.exp(-0.5 * wv * wv)
                        idx2_v[ring, pl.ds(m * L, L)] = dv

                    @pl.loop(0, K // L)
                    def _scale(r16):
                        w16 = ew_v[pl.ds(coff + r16 * L, L)]
                        for j in range(L):
                            s = w16[j]
                            r = r16 * L + j
                            for c8 in range(D // L):
                                bufs[slot][r, pl.ds(c8 * L, L)] = (
                                    bufs[slot][r, pl.ds(c8 * L, L)] * s)

                    pltpu.async_copy(bufs[slot], acc_sh.at[idx2_v.at[ring]],
                                     ssems[slot], add=True)

                    @pl.when(gc + NB < G)
                    def _prefetch():
                        pltpu.make_async_copy(bufs[slot],
                                              acc_sh.at[pl.ds(0, K)],
                                              ssems[slot]).wait()
                        pltpu.async_copy(xb.at[esrc_v.at[pl.ds(noff, K)]],
                                         bufs[slot], gsems[slot])

        for slot in range(NB):
            pltpu.make_async_copy(bufs[slot], acc_sh.at[pl.ds(0, K)],
                                  ssems[slot]).wait()

        plsc.subcore_barrier()

        @pl.loop(0, KCH)
        def _wb(k):
            c = sid + NS * k

            @pl.when(c < NCH)
            def _wbc():
                pltpu.sync_copy(acc_sh.at[pl.ds(c * ZCH, ZCH)], buf0)

                @pl.loop(0, ZCH // L)
                def _wn(r16):
                    dvec = c * ZCH + r16 * L + lax.iota(jnp.int32, 16)
                    nv = plsc.load_gather(
                        table_v,
                        [jnp.right_shift(dvec, 4), jnp.bitwise_and(dvec, 15)])
                    inv = 1.0 / (nv + 1e-8)
                    for j in range(L):
                        s = inv[j]
                        r = r16 * L + j
                        for c8 in range(D // L):
                            buf0[r, pl.ds(c8 * L, L)] = (
                                buf0[r, pl.ds(c8 * L, L)] * s)
                pltpu.sync_copy(buf0, out_hbm.at[b].at[pl.ds(c * ZCH, ZCH)])
        plsc.subcore_barrier()


def kernel(x, edge_index, weights):
    src1d = edge_index[0]
    dst1d = edge_index[1]
    mesh = plsc.VectorSubcoreMesh(core_axis_name="c", subcore_axis_name="s")
    f = pl.kernel(
        _body,
        out_type=jax.ShapeDtypeStruct((B, N_DST, D), jnp.float32),
        mesh=mesh,
        compiler_params=pltpu.CompilerParams(needs_layout_passes=False,
                                             use_tc_tiling_on_sc=False),
        scratch_types=[
            pltpu.VMEM((NEB * GRPK,), jnp.int32),
            pltpu.VMEM((NEB * GRPK,), jnp.int32),
            pltpu.VMEM((NEB * GRPK,), jnp.float32),
            pltpu.VMEM((NP // L, L), jnp.float32),
            pltpu.VMEM((SR, L), jnp.float32),
            pltpu.VMEM((NR // K, K), jnp.int32),
            pltpu.VMEM((NEB * GRP, K), jnp.int32),
            pltpu.VMEM((K, D), jnp.float32),
            pltpu.VMEM((K, D), jnp.float32),
            pltpu.VMEM_SHARED((NP // L, L), jnp.float32),
            pltpu.VMEM_SHARED((N_DST, D), jnp.float32),
            pltpu.SemaphoreType.DMA,
            pltpu.SemaphoreType.DMA,
            pltpu.SemaphoreType.DMA,
            pltpu.SemaphoreType.DMA,
            pltpu.SemaphoreType.DMA,
        ],
    )
    return f(x, src1d, dst1d, weights)


# speedup vs baseline: 68.7959x; 1.0752x over previous
import jax
import jax.numpy as jnp
from jax import lax
from jax.experimental import pallas as pl
from jax.experimental.pallas import tpu as pltpu
from jax.experimental.pallas import tpu_sc as plsc

N_DST = 10000
D = 128
B = 4
E = 320000
NS = 16
L = 16
TE = E // NS
K = 80
G = TE // K
GRP = 10
GRPK = GRP * K
NGRP = G // GRP
NEB = 3
NB = 3
NI2 = 8
NP = 10240
NR = NP // L
SR = NR // NS
ZCH = 80
NCH = N_DST // ZCH
KCH = -(-NCH // NS)


def _body(x_hbm, src_hbm, dst_hbm, w_hbm, out_hbm,
          esrc_v, edst_v, ew_v, table_v, tmp_v, idxr_v, idx2_v,
          buf0, buf1, buf2, norm_sh, acc_sh,
          esem, gsem0, gsem1, gsem2, ssem0, ssem1, ssem2):
    cid = lax.axis_index("c")
    sid = lax.axis_index("s")
    eb = sid * TE

    def issue_egroup(grp, with_src):
        off = eb + grp * GRPK
        dst_off = lax.rem(grp, NEB) * GRPK
        if with_src:
            pltpu.async_copy(src_hbm.at[pl.ds(off, GRPK)],
                             esrc_v.at[pl.ds(dst_off, GRPK)], esem)
        pltpu.async_copy(dst_hbm.at[pl.ds(off, GRPK)],
                         edst_v.at[pl.ds(dst_off, GRPK)], esem)
        pltpu.async_copy(w_hbm.at[pl.ds(off, GRPK)],
                         ew_v.at[pl.ds(dst_off, GRPK)], esem)

    def wait_egroup(with_src):
        if with_src:
            pltpu.make_async_copy(src_hbm.at[pl.ds(0, GRPK)],
                                  esrc_v.at[pl.ds(0, GRPK)], esem).wait()
        pltpu.make_async_copy(dst_hbm.at[pl.ds(0, GRPK)],
                              edst_v.at[pl.ds(0, GRPK)], esem).wait()
        pltpu.make_async_copy(w_hbm.at[pl.ds(0, GRPK)],
                              ew_v.at[pl.ds(0, GRPK)], esem).wait()

    @pl.loop(0, NR)
    def _zero_table(i):
        table_v[i] = jnp.zeros((L,), jnp.float32)

    for k in range(NR // K):
        for m in range(K // L):
            idxr_v[k, pl.ds(m * L, L)] = (lax.iota(jnp.int32, 16)
                                          + (k * K + m * L))

    @pl.loop(0, SR)
    def _zero_tmp(i):
        tmp_v[i] = jnp.zeros((L,), jnp.float32)

    for g in range(NEB):
        issue_egroup(g, False)

    @pl.loop(0, NGRP)
    def _p1(grp):
        ebuf = lax.rem(grp, NEB) * GRPK
        wait_egroup(False)

        @pl.loop(0, GRPK // L)
        def _s(m):
            off = ebuf + m * L
            wv = ew_v[pl.ds(off, L)]
            wg = jnp.exp(-0.5 * wv * wv)
            dv = edst_v[pl.ds(off, L)]
            plsc.addupdate_scatter(
                table_v, [jnp.right_shift(dv, 4), jnp.bitwise_and(dv, 15)],
                wg)

        nxt = grp + NEB

        @pl.when(nxt < NGRP)
        def _issue():
            issue_egroup(nxt, False)

    pltpu.sync_copy(tmp_v, norm_sh.at[pl.ds(sid * SR, SR)])
    plsc.subcore_barrier()
    for k in range(NR // K):
        pltpu.sync_copy(table_v.at[pl.ds(k * K, K)],
                        norm_sh.at[idxr_v.at[k]], add=True)
    plsc.subcore_barrier()
    pltpu.sync_copy(norm_sh, table_v)

    bufs = (buf0, buf1, buf2)
    gsems = (gsem0, gsem1, gsem2)
    ssems = (ssem0, ssem1, ssem2)

    RING = NEB * GRP

    @pl.loop(0, B // 2)
    def _batch(bi):
        b = cid * (B // 2) + bi
        xb = x_hbm.at[b]

        @pl.loop(0, ZCH)
        def _zz(r):
            for c8 in range(D // L):
                buf0[r, pl.ds(c8 * L, L)] = jnp.zeros((L,), jnp.float32)

        @pl.loop(0, KCH)
        def _z(k):
            c = sid + NS * k

            @pl.when(c < NCH)
            def _zc():
                pltpu.sync_copy(buf0, acc_sh.at[pl.ds(c * ZCH, ZCH)])
        plsc.subcore_barrier()

        issue_egroup(0, True)
        issue_egroup(1, True)
        wait_egroup(True)
        for slot in range(NB):
            pltpu.async_copy(xb.at[esrc_v.at[pl.ds(slot * K, K)]],
                             bufs[slot], gsems[slot])

        def _chunk(gc, slot):
            ring = lax.rem(gc, RING)
            coff = ring * K
            noff = lax.rem(gc + NB, RING) * K
            ri2 = lax.rem(gc, NI2)

            pltpu.make_async_copy(xb.at[pl.ds(0, K)], bufs[slot],
                                  gsems[slot]).wait()

            @pl.loop(0, K // L)
            def _norm(m):
                dv = edst_v[pl.ds(coff + m * L, L)]
                wv = ew_v[pl.ds(coff + m * L, L)]
                ew_v[pl.ds(coff + m * L, L)] = jnp.exp(-0.5 * wv * wv)
                idx2_v[ri2, pl.ds(m * L, L)] = dv

            @pl.loop(0, K // L)
            def _scale(r16):
                w16 = ew_v[pl.ds(coff + r16 * L, L)]
                for j in range(L):
                    s = w16[j]
                    r = r16 * L + j
                    for c8 in range(D // L):
                        bufs[slot][r, pl.ds(c8 * L, L)] = (
                            bufs[slot][r, pl.ds(c8 * L, L)] * s)

            pltpu.async_copy(bufs[slot], acc_sh.at[idx2_v.at[ri2]],
                             ssems[slot], add=True)

            @pl.when(gc + NB < G)
            def _prefetch():
                pltpu.make_async_copy(bufs[slot], acc_sh.at[pl.ds(0, K)],
                                      ssems[slot]).wait()
                pltpu.async_copy(xb.at[esrc_v.at[pl.ds(noff, K)]],
                                 bufs[slot], gsems[slot])

        GM = (G // NB) * NB

        @pl.loop(0, GM, step=NB)
        def _g2(cc0):
            for slot in range(NB):
                gc = cc0 + slot
                grp = lax.div(gc, GRP)

                @pl.when(lax.rem(gc, GRP) == 0)
                def _gev():
                    @pl.when(grp + 2 < NGRP)
                    def _issue():
                        issue_egroup(grp + 2, True)

                    @pl.when(grp + 1 < NGRP)
                    def _wait():
                        wait_egroup(True)
                _chunk(gc, slot)

        for gc in range(GM, G):
            _chunk(gc, gc % NB)

        for slot in range(NB):
            pltpu.make_async_copy(bufs[slot], acc_sh.at[pl.ds(0, K)],
                                  ssems[slot]).wait()

        plsc.subcore_barrier()

        @pl.loop(0, KCH)
        def _wb(k):
            c = sid + NS * k

            @pl.when(c < NCH)
            def _wbc():
                pltpu.sync_copy(acc_sh.at[pl.ds(c * ZCH, ZCH)], buf0)

                @pl.loop(0, ZCH // L)
                def _wn(r16):
                    dvec = c * ZCH + r16 * L + lax.iota(jnp.int32, 16)
                    nv = plsc.load_gather(
                        table_v,
                        [jnp.right_shift(dvec, 4), jnp.bitwise_and(dvec, 15)])
                    inv = 1.0 / (nv + 1e-8)
                    for j in range(L):
                        s = inv[j]
                        r = r16 * L + j
                        for c8 in range(D // L):
                            buf0[r, pl.ds(c8 * L, L)] = (
                                buf0[r, pl.ds(c8 * L, L)] * s)
                pltpu.sync_copy(buf0, out_hbm.at[b].at[pl.ds(c * ZCH, ZCH)])
        plsc.subcore_barrier()


def kernel(x, edge_index, weights):
    src1d = edge_index[0]
    dst1d = edge_index[1]
    mesh = plsc.VectorSubcoreMesh(core_axis_name="c", subcore_axis_name="s")
    f = pl.kernel(
        _body,
        out_type=jax.ShapeDtypeStruct((B, N_DST, D), jnp.float32),
        mesh=mesh,
        compiler_params=pltpu.CompilerParams(needs_layout_passes=False,
                                             use_tc_tiling_on_sc=False),
        scratch_types=[
            pltpu.VMEM((NEB * GRPK,), jnp.int32),
            pltpu.VMEM((NEB * GRPK,), jnp.int32),
            pltpu.VMEM((NEB * GRPK,), jnp.float32),
            pltpu.VMEM((NP // L, L), jnp.float32),
            pltpu.VMEM((SR, L), jnp.float32),
            pltpu.VMEM((NR // K, K), jnp.int32),
            pltpu.VMEM((NI2, K), jnp.int32),
            pltpu.VMEM((K, D), jnp.float32),
            pltpu.VMEM((K, D), jnp.float32),
            pltpu.VMEM((K, D), jnp.float32),
            pltpu.VMEM_SHARED((NP // L, L), jnp.float32),
            pltpu.VMEM_SHARED((N_DST, D), jnp.float32),
            pltpu.SemaphoreType.DMA,
            pltpu.SemaphoreType.DMA,
            pltpu.SemaphoreType.DMA,
            pltpu.SemaphoreType.DMA,
            pltpu.SemaphoreType.DMA,
            pltpu.SemaphoreType.DMA,
            pltpu.SemaphoreType.DMA,
        ],
    )
    return f(x, src1d, dst1d, weights)


# speedup vs baseline: 71.6701x; 1.0418x over previous
import jax
import jax.numpy as jnp
from jax import lax
from jax.experimental import pallas as pl
from jax.experimental.pallas import tpu as pltpu
from jax.experimental.pallas import tpu_sc as plsc

N_DST = 10000
D = 128
B = 4
E = 320000
NS = 16
L = 16
TE = E // NS
K = 80
G = TE // K
GRP = 10
GRPK = GRP * K
NGRP = G // GRP
NEB = 3
NB = 3
NI2 = 8
NP = 10240
NR = NP // L
SR = NR // NS
ZCH = 80
NCH = N_DST // ZCH
KCH = -(-NCH // NS)


def _body(x_hbm, src_hbm, dst_hbm, w_hbm, out_hbm,
          esrc_v, edst_v, ew_v, table_v, tmp_v, idxr_v, idx2_v,
          buf0, buf1, buf2, norm_sh, acc_sh,
          esem, gsem0, gsem1, gsem2, ssem0, ssem1, ssem2):
    cid = lax.axis_index("c")
    sid = lax.axis_index("s")
    eb = sid * TE

    def issue_egroup(grp, with_src):
        off = eb + grp * GRPK
        dst_off = lax.rem(grp, NEB) * GRPK
        if with_src:
            pltpu.async_copy(src_hbm.at[pl.ds(off, GRPK)],
                             esrc_v.at[pl.ds(dst_off, GRPK)], esem)
        pltpu.async_copy(dst_hbm.at[pl.ds(off, GRPK)],
                         edst_v.at[pl.ds(dst_off, GRPK)], esem)
        pltpu.async_copy(w_hbm.at[pl.ds(off, GRPK)],
                         ew_v.at[pl.ds(dst_off, GRPK)], esem)

    def wait_egroup(with_src):
        if with_src:
            pltpu.make_async_copy(src_hbm.at[pl.ds(0, GRPK)],
                                  esrc_v.at[pl.ds(0, GRPK)], esem).wait()
        pltpu.make_async_copy(dst_hbm.at[pl.ds(0, GRPK)],
                              edst_v.at[pl.ds(0, GRPK)], esem).wait()
        pltpu.make_async_copy(w_hbm.at[pl.ds(0, GRPK)],
                              ew_v.at[pl.ds(0, GRPK)], esem).wait()

    @pl.loop(0, NR)
    def _zero_table(i):
        table_v[i] = jnp.zeros((L,), jnp.float32)

    for k in range(NR // K):
        for m in range(K // L):
            idxr_v[k, pl.ds(m * L, L)] = (lax.iota(jnp.int32, 16)
                                          + (k * K + m * L))

    @pl.loop(0, SR)
    def _zero_tmp(i):
        tmp_v[i] = jnp.zeros((L,), jnp.float32)

    for g in range(NEB):
        issue_egroup(g, False)

    @pl.loop(0, NGRP)
    def _p1(grp):
        ebuf = lax.rem(grp, NEB) * GRPK
        wait_egroup(False)

        @pl.loop(0, GRPK // L)
        def _s(m):
            off = ebuf + m * L
            wv = ew_v[pl.ds(off, L)]
            wg = jnp.exp(-0.5 * wv * wv)
            dv = edst_v[pl.ds(off, L)]
            plsc.addupdate_scatter(
                table_v, [jnp.right_shift(dv, 4), jnp.bitwise_and(dv, 15)],
                wg)

        nxt = grp + NEB

        @pl.when(nxt < NGRP)
        def _issue():
            issue_egroup(nxt, False)

    pltpu.sync_copy(tmp_v, norm_sh.at[pl.ds(sid * SR, SR)])
    plsc.subcore_barrier()
    for k in range(NR // K):
        pltpu.sync_copy(table_v.at[pl.ds(k * K, K)],
                        norm_sh.at[idxr_v.at[k]], add=True)
    plsc.subcore_barrier()
    pltpu.sync_copy(norm_sh, table_v)

    bufs = (buf0, buf1, buf2)
    gsems = (gsem0, gsem1, gsem2)
    ssems = (ssem0, ssem1, ssem2)

    RING = NEB * GRP

    @pl.loop(0, B // 2)
    def _batch(bi):
        b = cid * (B // 2) + bi
        xb = x_hbm.at[b]

        @pl.loop(0, ZCH)
        def _zz(r):
            for c8 in range(D // L):
                buf0[r, pl.ds(c8 * L, L)] = jnp.zeros((L,), jnp.float32)

        @pl.loop(0, KCH)
        def _z(k):
            c = sid + NS * k

            @pl.when(c < NCH)
            def _zc():
                pltpu.sync_copy(buf0, acc_sh.at[pl.ds(c * ZCH, ZCH)])
        plsc.subcore_barrier()

        issue_egroup(0, True)
        issue_egroup(1, True)
        wait_egroup(True)
        for slot in range(2):
            pltpu.async_copy(xb.at[esrc_v.at[pl.ds(slot * K, K)]],
                             bufs[slot], gsems[slot])

        PF = 2

        def _chunk(gc, slot, drain):
            ring = lax.rem(gc, RING)
            coff = ring * K
            noff = lax.rem(gc + PF, RING) * K
            ri2 = lax.rem(gc, NI2)
            slot2 = (slot + PF) % NB

            pltpu.make_async_copy(xb.at[pl.ds(0, K)], bufs[slot],
                                  gsems[slot]).wait()

            @pl.loop(0, K // L)
            def _norm(m):
                dv = edst_v[pl.ds(coff + m * L, L)]
                wv = ew_v[pl.ds(coff + m * L, L)]
                ew_v[pl.ds(coff + m * L, L)] = jnp.exp(-0.5 * wv * wv)
                idx2_v[ri2, pl.ds(m * L, L)] = dv

            @pl.loop(0, K // L)
            def _scale(r16):
                w16 = ew_v[pl.ds(coff + r16 * L, L)]
                for j in range(L):
                    s = w16[j]
                    r = r16 * L + j
                    for c8 in range(D // L):
                        bufs[slot][r, pl.ds(c8 * L, L)] = (
                            bufs[slot][r, pl.ds(c8 * L, L)] * s)

            pltpu.async_copy(bufs[slot], acc_sh.at[idx2_v.at[ri2]],
                             ssems[slot], add=True)

            @pl.when(gc + PF < G)
            def _prefetch():
                if drain:
                    pltpu.make_async_copy(bufs[slot2], acc_sh.at[pl.ds(0, K)],
                                          ssems[slot2]).wait()
                pltpu.async_copy(xb.at[esrc_v.at[pl.ds(noff, K)]],
                                 bufs[slot2], gsems[slot2])

        GM = (G // NB) * NB

        issue_egroup(2, True)
        wait_egroup(True)
        _chunk(0, 0, False)
        _chunk(1, 1, True)
        _chunk(2, 2, True)

        @pl.loop(NB, GM, step=NB)
        def _g2(cc0):
            for slot in range(NB):
                gc = cc0 + slot
                grp = lax.div(gc, GRP)

                @pl.when(lax.rem(gc, GRP) == 0)
                def _gev():
                    @pl.when(grp + 2 < NGRP)
                    def _issue():
                        issue_egroup(grp + 2, True)

                    @pl.when(grp + 1 < NGRP)
                    def _wait():
                        wait_egroup(True)
                _chunk(gc, slot, True)

        for gc in range(GM, G):
            _chunk(gc, gc % NB, True)

        for slot in range(NB):
            pltpu.make_async_copy(bufs[slot], acc_sh.at[pl.ds(0, K)],
                                  ssems[slot]).wait()

        plsc.subcore_barrier()

        @pl.loop(0, KCH)
        def _wb(k):
            c = sid + NS * k

            @pl.when(c < NCH)
            def _wbc():
                pltpu.sync_copy(acc_sh.at[pl.ds(c * ZCH, ZCH)], buf0)

                @pl.loop(0, ZCH // L)
                def _wn(r16):
                    dvec = c * ZCH + r16 * L + lax.iota(jnp.int32, 16)
                    nv = plsc.load_gather(
                        table_v,
                        [jnp.right_shift(dvec, 4), jnp.bitwise_and(dvec, 15)])
                    inv = 1.0 / (nv + 1e-8)
                    for j in range(L):
                        s = inv[j]
                        r = r16 * L + j
                        for c8 in range(D // L):
                            buf0[r, pl.ds(c8 * L, L)] = (
                                buf0[r, pl.ds(c8 * L, L)] * s)
                pltpu.sync_copy(buf0, out_hbm.at[b].at[pl.ds(c * ZCH, ZCH)])
        plsc.subcore_barrier()


def kernel(x, edge_index, weights):
    src1d = edge_index[0]
    dst1d = edge_index[1]
    mesh = plsc.VectorSubcoreMesh(core_axis_name="c", subcore_axis_name="s")
    f = pl.kernel(
        _body,
        out_type=jax.ShapeDtypeStruct((B, N_DST, D), jnp.float32),
        mesh=mesh,
        compiler_params=pltpu.CompilerParams(needs_layout_passes=False,
                                             use_tc_tiling_on_sc=False),
        scratch_types=[
            pltpu.VMEM((NEB * GRPK,), jnp.int32),
            pltpu.VMEM((NEB * GRPK,), jnp.int32),
            pltpu.VMEM((NEB * GRPK,), jnp.float32),
            pltpu.VMEM((NP // L, L), jnp.float32),
            pltpu.VMEM((SR, L), jnp.float32),
            pltpu.VMEM((NR // K, K), jnp.int32),
            pltpu.VMEM((NI2, K), jnp.int32),
            pltpu.VMEM((K, D), jnp.float32),
            pltpu.VMEM((K, D), jnp.float32),
            pltpu.VMEM((K, D), jnp.float32),
            pltpu.VMEM_SHARED((NP // L, L), jnp.float32),
            pltpu.VMEM_SHARED((N_DST, D), jnp.float32),
            pltpu.SemaphoreType.DMA,
            pltpu.SemaphoreType.DMA,
            pltpu.SemaphoreType.DMA,
            pltpu.SemaphoreType.DMA,
            pltpu.SemaphoreType.DMA,
            pltpu.SemaphoreType.DMA,
            pltpu.SemaphoreType.DMA,
        ],
    )
    return f(x, src1d, dst1d, weights)
